# baseline (speedup 1.0000x reference)
import functools

import jax
import jax.numpy as jnp
from jax import lax
from jax.experimental import pallas as pl
from jax.experimental.pallas import tpu as pltpu
from jax.experimental.pallas import tpu_sc as plsc

_N = 10000
_E = 320000
_C = 128
_H = _C // 2
_K = 80
_CH_TOT = _E // _K
_NC = 2
_NS = 16
_NW = _NC * _NS
_CH_T = _CH_TOT // _NS
_CH_W = _CH_TOT // _NW
_NBUF = 5
_RPT = _N // _NS
_NACC = _N
_DEGW = 16


def _msg_body(tp3, src_t, dst_t, zer, out, src_v, dst_v, buf, acc, gsem, ssem):
    c = lax.axis_index("c")
    s = lax.axis_index("s")

    pltpu.sync_copy(src_t.at[s], src_v)
    pltpu.sync_copy(dst_t.at[s], dst_v)
    pltpu.sync_copy(zer, acc.at[pl.ds(s * _RPT, _RPT)])
    plsc.subcore_barrier()

    tp = tp3.at[c]

    def gather_start(j, b):
        pltpu.async_copy(tp.at[src_v.at[j]], buf.at[b], gsem.at[b])

    def gather_wait(j, b):
        pltpu.make_async_copy(tp.at[src_v.at[j]], buf.at[b], gsem.at[b]).wait()

    def scatter_start(j, b):
        pltpu.async_copy(buf.at[b], acc.at[dst_v.at[j]], ssem.at[b], add=True)

    def scatter_wait(j, b):
        pltpu.make_async_copy(buf.at[b], acc.at[dst_v.at[j]], ssem.at[b]).wait()

    for b in range(_NBUF):
        gather_start(b, b)

    def outer(i, carry):
        j0 = i * _NBUF
        for b in range(_NBUF):
            gather_wait(j0 + b, b)
            scatter_start(j0 + b, b)
        for b in range(_NBUF):
            scatter_wait(j0 + b, b)
            nj = j0 + b + _NBUF

            @pl.when(nj < _CH_T)
            def _():
                gather_start(nj, b)
        return carry

    lax.fori_loop(0, _CH_T // _NBUF, outer, 0)
    plsc.subcore_barrier()
    pltpu.sync_copy(acc.at[pl.ds(s * _RPT, _RPT)], out.at[c, s])


def _deg_body(dst_t, ones, zer, out, dst_v, ones_v, acc, sems):
    c = lax.axis_index("c")
    s = lax.axis_index("s")
    w = c * _NS + s

    pltpu.sync_copy(dst_t.at[w], dst_v)
    pltpu.sync_copy(ones, ones_v)
    pltpu.sync_copy(zer, acc.at[pl.ds(s * _RPT, _RPT)])
    plsc.subcore_barrier()

    def start(j, b):
        pltpu.async_copy(ones_v, acc.at[dst_v.at[j]], sems.at[b], add=True)

    def wait(b):
        pltpu.make_async_copy(ones_v, acc.at[dst_v.at[0]], sems.at[b]).wait()

    for b in range(_NBUF):
        start(b, b)

    def outer(i, carry):
        for b in range(_NBUF):
            j = i * _NBUF + b
            wait(b)
            nj = j + _NBUF

            @pl.when(nj < _CH_W)
            def _():
                start(nj, b)
        return carry

    lax.fori_loop(0, _CH_W // _NBUF, outer, 0)
    plsc.subcore_barrier()
    pltpu.sync_copy(acc.at[pl.ds(s * _RPT, _RPT)], out.at[c, s])


@functools.cache
def _sc_kernels():
    mesh = plsc.VectorSubcoreMesh(core_axis_name="c", subcore_axis_name="s",
                                  num_cores=_NC, num_subcores=_NS)
    params = pltpu.CompilerParams(use_tc_tiling_on_sc=False)
    msg = functools.partial(
        pl.kernel,
        out_type=jax.ShapeDtypeStruct((_NC, _NS, _RPT, _H), jnp.float32),
        mesh=mesh,
        compiler_params=params,
        scratch_types=[
            pltpu.VMEM((_CH_T, _K), jnp.int32),
            pltpu.VMEM((_CH_T, _K), jnp.int32),
            pltpu.VMEM((_NBUF, _K, _H), jnp.float32),
            pltpu.VMEM_SHARED((_NACC, _H), jnp.float32),
            pltpu.SemaphoreType.DMA((_NBUF,)),
            pltpu.SemaphoreType.DMA((_NBUF,)),
        ],
    )(_msg_body)
    deg = functools.partial(
        pl.kernel,
        out_type=jax.ShapeDtypeStruct((_NC, _NS, _RPT, _DEGW), jnp.float32),
        mesh=mesh,
        compiler_params=params,
        scratch_types=[
            pltpu.VMEM((_CH_W, _K), jnp.int32),
            pltpu.VMEM((_K, _DEGW), jnp.float32),
            pltpu.VMEM_SHARED((_NACC, _DEGW), jnp.float32),
            pltpu.SemaphoreType.DMA((_NBUF,)),
        ],
    )(_deg_body)
    return msg, deg


_BN = 512
_GRID = (pl.cdiv(_N, _BN),)


def _row_spec(w):
    return pl.BlockSpec((_BN, w), lambda i: (i, 0))


def _half_spec(w=None):
    return pl.BlockSpec((_NC, _BN, w or _H), lambda i: (0, i, 0))


def _full_spec(h, w):
    return pl.BlockSpec((h, w), lambda i: (0, 0))


def _relu(x):
    return jnp.maximum(x, 0.0)


def _mm(a, b):
    return jnp.dot(a, b, preferred_element_type=jnp.float32)


def _split3(o, res):
    o[0] = res[:, :_H]
    o[1] = res[:, _H:]


def _cat3(a3):
    return jnp.concatenate([a3[0], a3[1]], axis=1)


def _prologue_body(x, w_pre, b_pre, w_fc1, b_fc1, w_fc2, b_fc2,
                   wdt0, bd0, wdt1, bd1, wdt2, bd2, wf1t, bf1, wg0,
                   t0r_o, p0_o, p1_o, p2_o, pf_o):
    xx = x[...]
    h = _relu(_mm(xx, w_pre[...]) + b_pre[...])
    ni = _relu(_mm(h, w_fc1[...]) + b_fc1[...])
    h2 = _relu(_mm(h, w_fc2[...]) + b_fc2[...])
    _split3(t0r_o, _mm(h2, wg0[...]))
    p0_o[...] = _mm(ni, wdt0[...]) + bd0[...]
    p1_o[...] = _mm(ni, wdt1[...]) + bd1[...]
    p2_o[...] = _mm(ni, wdt2[...]) + bd2[...]
    pf_o[...] = _mm(ni, wf1t[...]) + bf1[...]


_prologue = pl.pallas_call(
    _prologue_body,
    grid=_GRID,
    in_specs=[_row_spec(_C)]
    + [_full_spec(_C, _C), _full_spec(1, _C)] * 3
    + [_full_spec(_C, _C), _full_spec(1, _C)] * 4
    + [_full_spec(_C, _C)],
    out_specs=[_half_spec()] + [_row_spec(_C)] * 4,
    out_shape=[jax.ShapeDtypeStruct((_NC, _N, _H), jnp.float32)]
    + [jax.ShapeDtypeStruct((_N, _C), jnp.float32)] * 4,
)


def _scale_body(t0r, d4, tp_o, dis_o):
    deg = d4[...][0, :, 0:1] + d4[...][1, :, 0:1] + 1.0
    dis = lax.rsqrt(deg)
    tp_o[0] = dis * t0r[...][0]
    tp_o[1] = dis * t0r[...][1]
    dis_o[...] = jnp.broadcast_to(dis, dis_o.shape)


_scale = pl.pallas_call(
    _scale_body,
    grid=_GRID,
    in_specs=[_half_spec(), _half_spec(_DEGW)],
    out_specs=[_half_spec(), _row_spec(_DEGW)],
    out_shape=[jax.ShapeDtypeStruct((_NC, _N, _H), jnp.float32),
               jax.ShapeDtypeStruct((_N, _DEGW), jnp.float32)],
)


def _layer_body(a3, tp3, dis, bg, wdb, pmat, wgn, tn_o):
    d = dis[...][:, 0:1]
    g = _relu(d * (_cat3(a3[...]) + _cat3(tp3[...])) + bg[...])
    h2 = _relu(_mm(g, wdb[...]) + pmat[...])
    _split3(tn_o, d * _mm(h2, wgn[...]))


_layer = pl.pallas_call(
    _layer_body,
    grid=_GRID,
    in_specs=[_half_spec(), _half_spec(), _row_spec(_DEGW),
              _full_spec(1, _C), _full_spec(_C, _C), _row_spec(_C),
              _full_spec(_C, _C)],
    out_specs=_half_spec(),
    out_shape=jax.ShapeDtypeStruct((_NC, _N, _H), jnp.float32),
)


def _final_body(a3, tp3, dis, bg, wdb, pmat, wf1b, pf, wf2, bf2, out_o):
    d = dis[...][:, 0:1]
    g = _relu(d * (_cat3(a3[...]) + _cat3(tp3[...])) + bg[...])
    h2 = _relu(_mm(g, wdb[...]) + pmat[...])
    f = _relu(_mm(h2, wf1b[...]) + pf[...])
    out_o[...] = _mm(f, wf2[...]) + bf2[...]


_final = pl.pallas_call(
    _final_body,
    grid=_GRID,
    in_specs=[_half_spec(), _half_spec(), _row_spec(_DEGW),
              _full_spec(1, _C), _full_spec(_C, _C), _row_spec(_C),
              _full_spec(_C, _C), _row_spec(_C),
              _full_spec(_C, 2), _full_spec(1, 2)],
    out_specs=_row_spec(2),
    out_shape=jax.ShapeDtypeStruct((_N, 2), jnp.float32),
)


@jax.jit
def kernel(x, edge_index, W_pre, b_pre, W_fc1, b_fc1, W_fc2, b_fc2,
           W_g0, b_g0, W_g1, b_g1, W_g2, b_g2,
           W_d0, b_d0, W_d1, b_d1, W_d2, b_d2,
           W_f1, b_f1, W_f2, b_f2):
    src_m = edge_index[0].reshape(_NS, _CH_T, _K)
    dst_m = edge_index[1].reshape(_NS, _CH_T, _K)
    dst_d = edge_index[1].reshape(_NW, _CH_W, _K)

    ones_col = jnp.zeros((_K, _DEGW), jnp.float32).at[:, 0].set(1.0)
    zer_deg = jnp.zeros((_RPT, _DEGW), jnp.float32)
    zer_msg = jnp.zeros((_RPT, _H), jnp.float32)

    msg_kernel, deg_kernel = _sc_kernels()
    deg4 = deg_kernel(dst_d, ones_col, zer_deg).reshape(_NC, _N, _DEGW)

    r = lambda b: b.reshape(1, -1)
    t0r, p0, p1, p2, pf = _prologue(
        x, W_pre, r(b_pre), W_fc1, r(b_fc1), W_fc2, r(b_fc2),
        W_d0[:_C], r(b_d0), W_d1[:_C], r(b_d1), W_d2[:_C], r(b_d2),
        W_f1[:_C], r(b_f1), W_g0)

    tp3, dis = _scale(t0r, deg4)

    bg = [b_g0, b_g1, b_g2]
    wdb = [W_d0[_C:], W_d1[_C:], W_d2[_C:]]
    pmat = [p0, p1, p2]
    wgn = [None, W_g1, W_g2]

    for i in range(2):
        a3 = msg_kernel(tp3, src_m, dst_m, zer_msg).reshape(_NC, _N, _H)
        tp3 = _layer(a3, tp3, dis, r(bg[i]), wdb[i], pmat[i], wgn[i + 1])

    a3 = msg_kernel(tp3, src_m, dst_m, zer_msg).reshape(_NC, _N, _H)
    out = _final(a3, tp3, dis, r(bg[2]), wdb[2], pmat[2],
                 W_f1[_C:], pf, W_f2, r(b_f2))
    return out


# speedup vs baseline: 1.1143x; 1.1143x over previous
import functools

import jax
import jax.numpy as jnp
from jax import lax
from jax.experimental import pallas as pl
from jax.experimental.pallas import tpu as pltpu
from jax.experimental.pallas import tpu_sc as plsc

_N = 10000
_E = 320000
_C = 128
_H = _C // 2
_K = 80
_CH_TOT = _E // _K
_NC = 2
_NS = 16
_NW = _NC * _NS
_CH_T = _CH_TOT // _NS
_CH_W = _CH_TOT // _NW
_NBUF = 5
_RPT = _N // _NS
_NACC = _N
_DEGW = 16


def _msg_body(tp3, src_t, dst_t, zer, out, src_v, dst_v, buf, acc, gsem, ssem):
    c = lax.axis_index("c")
    s = lax.axis_index("s")

    pltpu.sync_copy(src_t.at[s], src_v)
    pltpu.sync_copy(dst_t.at[s], dst_v)
    pltpu.sync_copy(zer, acc.at[pl.ds(s * _RPT, _RPT)])
    plsc.subcore_barrier()

    tp = tp3.at[c]

    def gather_start(j, b):
        pltpu.async_copy(tp.at[src_v.at[j]], buf.at[b], gsem.at[b])

    def gather_wait(j, b):
        pltpu.make_async_copy(tp.at[src_v.at[j]], buf.at[b], gsem.at[b]).wait()

    def scatter_start(j, b):
        pltpu.async_copy(buf.at[b], acc.at[dst_v.at[j]], ssem.at[b], add=True)

    def scatter_wait(j, b):
        pltpu.make_async_copy(buf.at[b], acc.at[dst_v.at[j]], ssem.at[b]).wait()

    for b in range(_NBUF):
        gather_start(b, b)

    def outer(i, carry):
        for b in range(_NBUF):
            j = i * _NBUF + b
            gather_wait(j, b)
            scatter_start(j, b)
            scatter_wait(j, b)
            nj = j + _NBUF

            @pl.when(nj < _CH_T)
            def _():
                gather_start(nj, b)
        return carry

    lax.fori_loop(0, _CH_T // _NBUF, outer, 0)
    plsc.subcore_barrier()
    pltpu.sync_copy(acc.at[pl.ds(s * _RPT, _RPT)], out.at[c, s])


def _deg_body(dst_t, ones, zer, out, dst_v, ones_v, acc, sems):
    c = lax.axis_index("c")
    s = lax.axis_index("s")
    w = c * _NS + s

    pltpu.sync_copy(dst_t.at[w], dst_v)
    pltpu.sync_copy(ones, ones_v)
    pltpu.sync_copy(zer, acc.at[pl.ds(s * _RPT, _RPT)])
    plsc.subcore_barrier()

    def start(j, b):
        pltpu.async_copy(ones_v, acc.at[dst_v.at[j]], sems.at[b], add=True)

    def wait(b):
        pltpu.make_async_copy(ones_v, acc.at[dst_v.at[0]], sems.at[b]).wait()

    for b in range(_NBUF):
        start(b, b)

    def outer(i, carry):
        for b in range(_NBUF):
            j = i * _NBUF + b
            wait(b)
            nj = j + _NBUF

            @pl.when(nj < _CH_W)
            def _():
                start(nj, b)
        return carry

    lax.fori_loop(0, _CH_W // _NBUF, outer, 0)
    plsc.subcore_barrier()
    pltpu.sync_copy(acc.at[pl.ds(s * _RPT, _RPT)], out.at[c, s])


@functools.cache
def _sc_kernels():
    mesh = plsc.VectorSubcoreMesh(core_axis_name="c", subcore_axis_name="s",
                                  num_cores=_NC, num_subcores=_NS)
    params = pltpu.CompilerParams(use_tc_tiling_on_sc=False)
    msg = functools.partial(
        pl.kernel,
        out_type=jax.ShapeDtypeStruct((_NC, _NS, _RPT, _H), jnp.float32),
        mesh=mesh,
        compiler_params=params,
        scratch_types=[
            pltpu.VMEM((_CH_T, _K), jnp.int32),
            pltpu.VMEM((_CH_T, _K), jnp.int32),
            pltpu.VMEM((_NBUF, _K, _H), jnp.float32),
            pltpu.VMEM_SHARED((_NACC, _H), jnp.float32),
            pltpu.SemaphoreType.DMA((_NBUF,)),
            pltpu.SemaphoreType.DMA((_NBUF,)),
        ],
    )(_msg_body)
    deg = functools.partial(
        pl.kernel,
        out_type=jax.ShapeDtypeStruct((_NC, _NS, _RPT, _DEGW), jnp.float32),
        mesh=mesh,
        compiler_params=params,
        scratch_types=[
            pltpu.VMEM((_CH_W, _K), jnp.int32),
            pltpu.VMEM((_K, _DEGW), jnp.float32),
            pltpu.VMEM_SHARED((_NACC, _DEGW), jnp.float32),
            pltpu.SemaphoreType.DMA((_NBUF,)),
        ],
    )(_deg_body)
    return msg, deg


_BN = 512
_GRID = (pl.cdiv(_N, _BN),)


def _row_spec(w):
    return pl.BlockSpec((_BN, w), lambda i: (i, 0))


def _half_spec(w=None):
    return pl.BlockSpec((_NC, _BN, w or _H), lambda i: (0, i, 0))


def _full_spec(h, w):
    return pl.BlockSpec((h, w), lambda i: (0, 0))


def _relu(x):
    return jnp.maximum(x, 0.0)


def _mm(a, b):
    return jnp.dot(a, b, preferred_element_type=jnp.float32)


def _split3(o, res):
    o[0] = res[:, :_H]
    o[1] = res[:, _H:]


def _cat3(a3):
    return jnp.concatenate([a3[0], a3[1]], axis=1)


def _prologue_body(x, w_pre, b_pre, w_fc1, b_fc1, w_fc2, b_fc2,
                   wdt0, bd0, wdt1, bd1, wdt2, bd2, wf1t, bf1,
                   h2_o, p0_o, p1_o, p2_o, pf_o):
    xx = x[...]
    h = _relu(_mm(xx, w_pre[...]) + b_pre[...])
    ni = _relu(_mm(h, w_fc1[...]) + b_fc1[...])
    h2_o[...] = _relu(_mm(h, w_fc2[...]) + b_fc2[...])
    p0_o[...] = _mm(ni, wdt0[...]) + bd0[...]
    p1_o[...] = _mm(ni, wdt1[...]) + bd1[...]
    p2_o[...] = _mm(ni, wdt2[...]) + bd2[...]
    pf_o[...] = _mm(ni, wf1t[...]) + bf1[...]


_prologue = pl.pallas_call(
    _prologue_body,
    grid=_GRID,
    in_specs=[_row_spec(_C)]
    + [_full_spec(_C, _C), _full_spec(1, _C)] * 3
    + [_full_spec(_C, _C), _full_spec(1, _C)] * 4,
    out_specs=[_row_spec(_C)] * 5,
    out_shape=[jax.ShapeDtypeStruct((_N, _C), jnp.float32)] * 5,
)


def _scale_body(h2, d4, wg0, tp_o, dis_o):
    deg = d4[...][0, :, 0:1] + d4[...][1, :, 0:1] + 1.0
    dis = lax.rsqrt(deg)
    _split3(tp_o, dis * _mm(h2[...], wg0[...]))
    dis_o[...] = jnp.broadcast_to(dis, dis_o.shape)


_scale = pl.pallas_call(
    _scale_body,
    grid=_GRID,
    in_specs=[_row_spec(_C), _half_spec(_DEGW), _full_spec(_C, _C)],
    out_specs=[_half_spec(), _row_spec(_DEGW)],
    out_shape=[jax.ShapeDtypeStruct((_NC, _N, _H), jnp.float32),
               jax.ShapeDtypeStruct((_N, _DEGW), jnp.float32)],
)


def _layer_body(a3, tp3, dis, bg, wdb, pmat, wgn, tn_o):
    d = dis[...][:, 0:1]
    g = _relu(d * (_cat3(a3[...]) + _cat3(tp3[...])) + bg[...])
    h2 = _relu(_mm(g, wdb[...]) + pmat[...])
    _split3(tn_o, d * _mm(h2, wgn[...]))


_layer = pl.pallas_call(
    _layer_body,
    grid=_GRID,
    in_specs=[_half_spec(), _half_spec(), _row_spec(_DEGW),
              _full_spec(1, _C), _full_spec(_C, _C), _row_spec(_C),
              _full_spec(_C, _C)],
    out_specs=_half_spec(),
    out_shape=jax.ShapeDtypeStruct((_NC, _N, _H), jnp.float32),
)


def _final_body(a3, tp3, dis, bg, wdb, pmat, wf1b, pf, wf2, bf2, out_o):
    d = dis[...][:, 0:1]
    g = _relu(d * (_cat3(a3[...]) + _cat3(tp3[...])) + bg[...])
    h2 = _relu(_mm(g, wdb[...]) + pmat[...])
    f = _relu(_mm(h2, wf1b[...]) + pf[...])
    out_o[...] = _mm(f, wf2[...]) + bf2[...]


_final = pl.pallas_call(
    _final_body,
    grid=_GRID,
    in_specs=[_half_spec(), _half_spec(), _row_spec(_DEGW),
              _full_spec(1, _C), _full_spec(_C, _C), _row_spec(_C),
              _full_spec(_C, _C), _row_spec(_C),
              _full_spec(_C, 2), _full_spec(1, 2)],
    out_specs=_row_spec(2),
    out_shape=jax.ShapeDtypeStruct((_N, 2), jnp.float32),
)


@jax.jit
def kernel(x, edge_index, W_pre, b_pre, W_fc1, b_fc1, W_fc2, b_fc2,
           W_g0, b_g0, W_g1, b_g1, W_g2, b_g2,
           W_d0, b_d0, W_d1, b_d1, W_d2, b_d2,
           W_f1, b_f1, W_f2, b_f2):
    src_m = edge_index[0].reshape(_NS, _CH_T, _K)
    dst_m = edge_index[1].reshape(_NS, _CH_T, _K)
    dst_d = edge_index[1].reshape(_NW, _CH_W, _K)

    ones_col = jnp.zeros((_K, _DEGW), jnp.float32).at[:, 0].set(1.0)
    zer_deg = jnp.zeros((_RPT, _DEGW), jnp.float32)
    zer_msg = jnp.zeros((_RPT, _H), jnp.float32)

    msg_kernel, deg_kernel = _sc_kernels()
    deg4 = deg_kernel(dst_d, ones_col, zer_deg).reshape(_NC, _N, _DEGW)

    r = lambda b: b.reshape(1, -1)
    h2, p0, p1, p2, pf = _prologue(
        x, W_pre, r(b_pre), W_fc1, r(b_fc1), W_fc2, r(b_fc2),
        W_d0[:_C], r(b_d0), W_d1[:_C], r(b_d1), W_d2[:_C], r(b_d2),
        W_f1[:_C], r(b_f1))

    tp3, dis = _scale(h2, deg4, W_g0)

    bg = [b_g0, b_g1, b_g2]
    wdb = [W_d0[_C:], W_d1[_C:], W_d2[_C:]]
    pmat = [p0, p1, p2]
    wgn = [None, W_g1, W_g2]

    for i in range(2):
        a3 = msg_kernel(tp3, src_m, dst_m, zer_msg).reshape(_NC, _N, _H)
        tp3 = _layer(a3, tp3, dis, r(bg[i]), wdb[i], pmat[i], wgn[i + 1])

    a3 = msg_kernel(tp3, src_m, dst_m, zer_msg).reshape(_NC, _N, _H)
    out = _final(a3, tp3, dis, r(bg[2]), wdb[2], pmat[2],
                 W_f1[_C:], pf, W_f2, r(b_f2))
    return out


# speedup vs baseline: 1.2143x; 1.0898x over previous
import functools

import jax
import jax.numpy as jnp
from jax import lax
from jax.experimental import pallas as pl
from jax.experimental.pallas import tpu as pltpu
from jax.experimental.pallas import tpu_sc as plsc

_N = 10000
_E = 320000
_C = 128
_H = _C // 2
_K = 80
_CH_TOT = _E // _K
_NC = 2
_NS = 16
_NW = _NC * _NS
_CH_T = _CH_TOT // _NS
_CH_W = _CH_TOT // _NW
_NBUF = 5
_RPT = _N // _NS
_TR = 626
_NACC = _NS * _TR
_OROWS = _TR * 64 // 128
_P = _NACC // 2
_DEGW = 16


def _msg_body(tp3, src_t, dst_t, zer, out, src_v, dst_v, buf, acc, gsem, ssem):
    c = lax.axis_index("c")
    s = lax.axis_index("s")

    pltpu.sync_copy(src_t.at[s], src_v)
    pltpu.sync_copy(dst_t.at[s], dst_v)
    pltpu.sync_copy(zer, acc.at[pl.ds(s * _TR, _TR)])
    plsc.subcore_barrier()

    tp = tp3.at[c]

    def gather_start(j, b):
        pltpu.async_copy(tp.at[src_v.at[j]], buf.at[b], gsem.at[b])

    def gather_wait(j, b):
        pltpu.make_async_copy(tp.at[src_v.at[j]], buf.at[b], gsem.at[b]).wait()

    def scatter_start(j, b):
        pltpu.async_copy(buf.at[b], acc.at[dst_v.at[j]], ssem.at[b], add=True)

    def scatter_wait(j, b):
        pltpu.make_async_copy(buf.at[b], acc.at[dst_v.at[j]], ssem.at[b]).wait()

    for b in range(_NBUF):
        gather_start(b, b)

    def outer(i, carry):
        for b in range(_NBUF):
            j = i * _NBUF + b
            gather_wait(j, b)
            scatter_start(j, b)
            scatter_wait(j, b)
            nj = j + _NBUF

            @pl.when(nj < _CH_T)
            def _():
                gather_start(nj, b)
        return carry

    lax.fori_loop(0, _CH_T // _NBUF, outer, 0)
    plsc.subcore_barrier()
    pltpu.sync_copy(acc.at[pl.ds(s * _RPT, _RPT)], out.at[c, s])


def _deg_body(dst_t, ones, zer, out, dst_v, ones_v, acc, sems):
    c = lax.axis_index("c")
    s = lax.axis_index("s")
    w = c * _NS + s

    pltpu.sync_copy(dst_t.at[w], dst_v)
    pltpu.sync_copy(ones, ones_v)
    pltpu.sync_copy(zer, acc.at[pl.ds(s * _RPT, _RPT)])
    plsc.subcore_barrier()

    def start(j, b):
        pltpu.async_copy(ones_v, acc.at[dst_v.at[j]], sems.at[b], add=True)

    def wait(b):
        pltpu.make_async_copy(ones_v, acc.at[dst_v.at[0]], sems.at[b]).wait()

    for b in range(_NBUF):
        start(b, b)

    def outer(i, carry):
        for b in range(_NBUF):
            j = i * _NBUF + b
            wait(b)
            nj = j + _NBUF

            @pl.when(nj < _CH_W)
            def _():
                start(nj, b)
        return carry

    lax.fori_loop(0, _CH_W // _NBUF, outer, 0)
    plsc.subcore_barrier()
    pltpu.sync_copy(acc.at[pl.ds(s * _RPT, _RPT)], out.at[c, s])


@functools.cache
def _sc_kernels():
    mesh = plsc.VectorSubcoreMesh(core_axis_name="c", subcore_axis_name="s",
                                  num_cores=_NC, num_subcores=_NS)
    params = pltpu.CompilerParams(use_tc_tiling_on_sc=False)
    msg = functools.partial(
        pl.kernel,
        out_type=jax.ShapeDtypeStruct((_NC, _NS, _RPT, _H), jnp.float32),
        mesh=mesh,
        compiler_params=params,
        scratch_types=[
            pltpu.VMEM((_CH_T, _K), jnp.int32),
            pltpu.VMEM((_CH_T, _K), jnp.int32),
            pltpu.VMEM((_NBUF, _K, _H), jnp.float32),
            pltpu.VMEM_SHARED((_NACC, _H), jnp.float32),
            pltpu.SemaphoreType.DMA((_NBUF,)),
            pltpu.SemaphoreType.DMA((_NBUF,)),
        ],
    )(_msg_body)
    deg = functools.partial(
        pl.kernel,
        out_type=jax.ShapeDtypeStruct((_NC, _NS, _RPT, _DEGW), jnp.float32),
        mesh=mesh,
        compiler_params=params,
        scratch_types=[
            pltpu.VMEM((_CH_W, _K), jnp.int32),
            pltpu.VMEM((_K, _DEGW), jnp.float32),
            pltpu.VMEM_SHARED((_NACC, _DEGW), jnp.float32),
            pltpu.SemaphoreType.DMA((_NBUF,)),
        ],
    )(_deg_body)
    return msg, deg


_BN = 512
_GRID = (pl.cdiv(_N, _BN),)


def _row_spec(w):
    return pl.BlockSpec((_BN, w), lambda i: (i, 0))


def _half_spec(w=None):
    return pl.BlockSpec((_NC, _BN, w or _H), lambda i: (0, i, 0))


def _pack_spec():
    return pl.BlockSpec((_NC, _BN // 2, _C), lambda i: (0, i, 0))


def _full_spec(h, w):
    return pl.BlockSpec((h, w), lambda i: (0, 0))


def _relu(x):
    return jnp.maximum(x, 0.0)


def _mm(a, b):
    return jnp.dot(a, b, preferred_element_type=jnp.float32)


def _pack(res):
    t = res.reshape(_BN // 2, 2, _NC, _H)
    return t.transpose(2, 0, 1, 3).reshape(_NC, _BN // 2, _C)


def _unpack(blk):
    t = blk.reshape(_NC, _BN // 2, 2, _H)
    return t.transpose(1, 2, 0, 3).reshape(_BN, _C)


def _prologue_body(x, w_pre, b_pre, w_fc1, b_fc1, w_fc2, b_fc2,
                   wdt0, bd0, wdt1, bd1, wdt2, bd2, wf1t, bf1,
                   h2_o, p0_o, p1_o, p2_o, pf_o):
    xx = x[...]
    h = _relu(_mm(xx, w_pre[...]) + b_pre[...])
    ni = _relu(_mm(h, w_fc1[...]) + b_fc1[...])
    h2_o[...] = _relu(_mm(h, w_fc2[...]) + b_fc2[...])
    p0_o[...] = _mm(ni, wdt0[...]) + bd0[...]
    p1_o[...] = _mm(ni, wdt1[...]) + bd1[...]
    p2_o[...] = _mm(ni, wdt2[...]) + bd2[...]
    pf_o[...] = _mm(ni, wf1t[...]) + bf1[...]


_prologue = pl.pallas_call(
    _prologue_body,
    grid=_GRID,
    in_specs=[_row_spec(_C)]
    + [_full_spec(_C, _C), _full_spec(1, _C)] * 3
    + [_full_spec(_C, _C), _full_spec(1, _C)] * 4,
    out_specs=[_row_spec(_C)] * 5,
    out_shape=[jax.ShapeDtypeStruct((_N, _C), jnp.float32)] * 5,
)


def _scale_body(h2, d4, wg0, tp_o, dis_o):
    deg = d4[...][0, :, 0:1] + d4[...][1, :, 0:1] + 1.0
    dis = lax.rsqrt(deg)
    tp_o[...] = _pack(dis * _mm(h2[...], wg0[...]))
    dis_o[...] = jnp.broadcast_to(dis, dis_o.shape)


_scale = pl.pallas_call(
    _scale_body,
    grid=_GRID,
    in_specs=[_row_spec(_C), _half_spec(_DEGW), _full_spec(_C, _C)],
    out_specs=[_pack_spec(), _row_spec(_DEGW)],
    out_shape=[jax.ShapeDtypeStruct((_NC, _P, _C), jnp.float32),
               jax.ShapeDtypeStruct((_N, _DEGW), jnp.float32)],
)


def _layer_body(a3, tp3, dis, bg, wdb, pmat, wgn, tn_o):
    d = dis[...][:, 0:1]
    g = _relu(d * (_unpack(a3[...]) + _unpack(tp3[...])) + bg[...])
    h2 = _relu(_mm(g, wdb[...]) + pmat[...])
    tn_o[...] = _pack(d * _mm(h2, wgn[...]))


_layer = pl.pallas_call(
    _layer_body,
    grid=_GRID,
    in_specs=[_pack_spec(), _pack_spec(), _row_spec(_DEGW),
              _full_spec(1, _C), _full_spec(_C, _C), _row_spec(_C),
              _full_spec(_C, _C)],
    out_specs=_pack_spec(),
    out_shape=jax.ShapeDtypeStruct((_NC, _P, _C), jnp.float32),
)


def _final_body(a3, tp3, dis, bg, wdb, pmat, wf1b, pf, wf2, bf2, out_o):
    d = dis[...][:, 0:1]
    g = _relu(d * (_unpack(a3[...]) + _unpack(tp3[...])) + bg[...])
    h2 = _relu(_mm(g, wdb[...]) + pmat[...])
    f = _relu(_mm(h2, wf1b[...]) + pf[...])
    out_o[...] = _mm(f, wf2[...]) + bf2[...]


_final = pl.pallas_call(
    _final_body,
    grid=_GRID,
    in_specs=[_pack_spec(), _pack_spec(), _row_spec(_DEGW),
              _full_spec(1, _C), _full_spec(_C, _C), _row_spec(_C),
              _full_spec(_C, _C), _row_spec(_C),
              _full_spec(_C, 2), _full_spec(1, 2)],
    out_specs=_row_spec(2),
    out_shape=jax.ShapeDtypeStruct((_N, 2), jnp.float32),
)


@jax.jit
def kernel(x, edge_index, W_pre, b_pre, W_fc1, b_fc1, W_fc2, b_fc2,
           W_g0, b_g0, W_g1, b_g1, W_g2, b_g2,
           W_d0, b_d0, W_d1, b_d1, W_d2, b_d2,
           W_f1, b_f1, W_f2, b_f2):
    src_m = edge_index[0].reshape(_NS, _CH_T, _K)
    dst_m = edge_index[1].reshape(_NS, _CH_T, _K)
    dst_d = edge_index[1].reshape(_NW, _CH_W, _K)

    ones_col = jnp.zeros((_K, _DEGW), jnp.float32).at[:, 0].set(1.0)
    zer_deg = jnp.zeros((_RPT, _DEGW), jnp.float32)
    zer_msg = jnp.zeros((_TR, _H), jnp.float32)

    msg_kernel, deg_kernel = _sc_kernels()
    deg4 = deg_kernel(dst_d, ones_col, zer_deg).reshape(_NC, _N, _DEGW)

    r = lambda b: b.reshape(1, -1)
    h2, p0, p1, p2, pf = _prologue(
        x, W_pre, r(b_pre), W_fc1, r(b_fc1), W_fc2, r(b_fc2),
        W_d0[:_C], r(b_d0), W_d1[:_C], r(b_d1), W_d2[:_C], r(b_d2),
        W_f1[:_C], r(b_f1))

    tp3, dis = _scale(h2, deg4, W_g0)

    bg = [b_g0, b_g1, b_g2]
    wdb = [W_d0[_C:], W_d1[_C:], W_d2[_C:]]
    pmat = [p0, p1, p2]
    wgn = [None, W_g1, W_g2]

    v = lambda t: t.reshape(_NC, _NACC, _H)
    pk = lambda a: a.reshape(_NC, _N // 2, _C)

    for i in range(2):
        a3 = pk(msg_kernel(v(tp3), src_m, dst_m, zer_msg))
        tp3 = _layer(a3, tp3, dis, r(bg[i]), wdb[i], pmat[i], wgn[i + 1])

    a3 = pk(msg_kernel(v(tp3), src_m, dst_m, zer_msg))
    out = _final(a3, tp3, dis, r(bg[2]), wdb[2], pmat[2],
                 W_f1[_C:], pf, W_f2, r(b_f2))
    return out


# speedup vs baseline: 1.2624x; 1.0396x over previous
import functools

import jax
import jax.numpy as jnp
from jax import lax
from jax.experimental import pallas as pl
from jax.experimental.pallas import tpu as pltpu
from jax.experimental.pallas import tpu_sc as plsc

_N = 10000
_E = 320000
_C = 128
_H = _C // 2
_K = 80
_CH_TOT = _E // _K
_NC = 2
_NS = 16
_NW = _NC * _NS
_CH_T = _CH_TOT // _NS
_CH_W = _CH_TOT // _NW
_NBUF = 5
_RPT = _N // _NS
_TR = 626
_NACC = _NS * _TR
_OROWS = _TR * 64 // 128
_P = _NACC // 2
_DEGW = 16


def _msg_body(tp3, src_t, dst_t, zer, out, src_v, dst_v, buf, acc, gsem, ssem):
    c = lax.axis_index("c")
    s = lax.axis_index("s")

    pltpu.sync_copy(src_t.at[s], src_v)
    pltpu.sync_copy(dst_t.at[s], dst_v)
    pltpu.sync_copy(zer, acc.at[pl.ds(s * _TR, _TR)])
    plsc.subcore_barrier()

    tp = tp3.at[c]

    def gather_start(j, b):
        pltpu.async_copy(tp.at[src_v.at[j]], buf.at[b], gsem.at[b])

    def gather_wait(j, b):
        pltpu.make_async_copy(tp.at[src_v.at[j]], buf.at[b], gsem.at[b]).wait()

    def scatter_start(j, b):
        pltpu.async_copy(buf.at[b], acc.at[dst_v.at[j]], ssem.at[b], add=True)

    def scatter_wait(j, b):
        pltpu.make_async_copy(buf.at[b], acc.at[dst_v.at[j]], ssem.at[b]).wait()

    for b in range(_NBUF):
        gather_start(b, b)

    def outer(i, carry):
        for b in range(_NBUF):
            j = i * _NBUF + b
            gather_wait(j, b)
            scatter_start(j, b)
            scatter_wait(j, b)
            nj = j + _NBUF

            @pl.when(nj < _CH_T)
            def _():
                gather_start(nj, b)
        return carry

    lax.fori_loop(0, _CH_T // _NBUF, outer, 0)
    plsc.subcore_barrier()
    pltpu.sync_copy(acc.at[pl.ds(s * _RPT, _RPT)], out.at[c, s])


def _deg_body(dst_t, ones, zer, out, dst_v, ones_v, acc, sems):
    c = lax.axis_index("c")
    s = lax.axis_index("s")

    pltpu.sync_copy(dst_t.at[s].at[pl.ds(c * _CH_W, _CH_W)], dst_v)
    pltpu.sync_copy(ones, ones_v)
    pltpu.sync_copy(zer, acc.at[pl.ds(s * _RPT, _RPT)])
    plsc.subcore_barrier()

    def start(j, b):
        pltpu.async_copy(ones_v, acc.at[dst_v.at[j]], sems.at[b], add=True)

    def wait(b):
        pltpu.make_async_copy(ones_v, acc.at[dst_v.at[0]], sems.at[b]).wait()

    for b in range(_NBUF):
        start(b, b)

    def outer(i, carry):
        for b in range(_NBUF):
            j = i * _NBUF + b
            wait(b)
            nj = j + _NBUF

            @pl.when(nj < _CH_W)
            def _():
                start(nj, b)
        return carry

    lax.fori_loop(0, _CH_W // _NBUF, outer, 0)
    plsc.subcore_barrier()
    pltpu.sync_copy(acc.at[pl.ds(s * _RPT, _RPT)], out.at[c, s])


@functools.cache
def _sc_kernels():
    mesh = plsc.VectorSubcoreMesh(core_axis_name="c", subcore_axis_name="s",
                                  num_cores=_NC, num_subcores=_NS)
    params = pltpu.CompilerParams(use_tc_tiling_on_sc=False)
    msg = functools.partial(
        pl.kernel,
        out_type=jax.ShapeDtypeStruct((_NC, _NS, _RPT, _H), jnp.float32),
        mesh=mesh,
        compiler_params=params,
        scratch_types=[
            pltpu.VMEM((_CH_T, _K), jnp.int32),
            pltpu.VMEM((_CH_T, _K), jnp.int32),
            pltpu.VMEM((_NBUF, _K, _H), jnp.float32),
            pltpu.VMEM_SHARED((_NACC, _H), jnp.float32),
            pltpu.SemaphoreType.DMA((_NBUF,)),
            pltpu.SemaphoreType.DMA((_NBUF,)),
        ],
    )(_msg_body)
    deg = functools.partial(
        pl.kernel,
        out_type=jax.ShapeDtypeStruct((_NC, _NS, _RPT, _DEGW), jnp.float32),
        mesh=mesh,
        compiler_params=params,
        scratch_types=[
            pltpu.VMEM((_CH_W, _K), jnp.int32),
            pltpu.VMEM((_K, _DEGW), jnp.float32),
            pltpu.VMEM_SHARED((_NACC, _DEGW), jnp.float32),
            pltpu.SemaphoreType.DMA((_NBUF,)),
        ],
    )(_deg_body)
    return msg, deg


_BN = 1024
_GRID = (pl.cdiv(_N, _BN),)


def _row_spec(w):
    return pl.BlockSpec((_BN, w), lambda i: (i, 0))


def _half_spec(w=None):
    return pl.BlockSpec((_NC, _BN, w or _H), lambda i: (0, i, 0))


def _pack_spec():
    return pl.BlockSpec((_NC, _BN // 2, _C), lambda i: (0, i, 0))


def _full_spec(h, w):
    return pl.BlockSpec((h, w), lambda i: (0, 0))


def _relu(x):
    return jnp.maximum(x, 0.0)


def _mm(a, b):
    return jnp.dot(a, b, preferred_element_type=jnp.float32)


def _pack(res):
    t = res.reshape(_BN // 2, 2, _NC, _H)
    return t.transpose(2, 0, 1, 3).reshape(_NC, _BN // 2, _C)


def _unpack(blk):
    t = blk.reshape(_NC, _BN // 2, 2, _H)
    return t.transpose(1, 2, 0, 3).reshape(_BN, _C)


def _prologue_body(x, w_pre, b_pre, w_fc1, b_fc1, w_fc2, b_fc2,
                   wdt0, bd0, wdt1, bd1, wdt2, bd2, wf1t, bf1,
                   h2_o, p0_o, p1_o, p2_o, pf_o):
    xx = x[...]
    h = _relu(_mm(xx, w_pre[...]) + b_pre[...])
    ni = _relu(_mm(h, w_fc1[...]) + b_fc1[...])
    h2_o[...] = _relu(_mm(h, w_fc2[...]) + b_fc2[...])
    p0_o[...] = _mm(ni, wdt0[...]) + bd0[...]
    p1_o[...] = _mm(ni, wdt1[...]) + bd1[...]
    p2_o[...] = _mm(ni, wdt2[...]) + bd2[...]
    pf_o[...] = _mm(ni, wf1t[...]) + bf1[...]


_prologue = pl.pallas_call(
    _prologue_body,
    grid=_GRID,
    in_specs=[_row_spec(_C)]
    + [_full_spec(_C, _C), _full_spec(1, _C)] * 3
    + [_full_spec(_C, _C), _full_spec(1, _C)] * 4,
    out_specs=[_row_spec(_C)] * 5,
    out_shape=[jax.ShapeDtypeStruct((_N, _C), jnp.float32)] * 5,
)


def _scale_body(h2, d4, wg0, tp_o, dis_o):
    deg = d4[...][0, :, 0:1] + d4[...][1, :, 0:1] + 1.0
    dis = lax.rsqrt(deg)
    tp_o[...] = _pack(dis * _mm(h2[...], wg0[...]))
    dis_o[...] = jnp.broadcast_to(dis, dis_o.shape)


_scale = pl.pallas_call(
    _scale_body,
    grid=_GRID,
    in_specs=[_row_spec(_C), _half_spec(_DEGW), _full_spec(_C, _C)],
    out_specs=[_pack_spec(), _row_spec(_DEGW)],
    out_shape=[jax.ShapeDtypeStruct((_NC, _P, _C), jnp.float32),
               jax.ShapeDtypeStruct((_N, _DEGW), jnp.float32)],
)


def _layer_body(a3, tp3, dis, bg, wdb, pmat, wgn, tn_o):
    d = dis[...][:, 0:1]
    g = _relu(d * (_unpack(a3[...]) + _unpack(tp3[...])) + bg[...])
    h2 = _relu(_mm(g, wdb[...]) + pmat[...])
    tn_o[...] = _pack(d * _mm(h2, wgn[...]))


_layer = pl.pallas_call(
    _layer_body,
    grid=_GRID,
    in_specs=[_pack_spec(), _pack_spec(), _row_spec(_DEGW),
              _full_spec(1, _C), _full_spec(_C, _C), _row_spec(_C),
              _full_spec(_C, _C)],
    out_specs=_pack_spec(),
    out_shape=jax.ShapeDtypeStruct((_NC, _P, _C), jnp.float32),
)


def _final_body(a3, tp3, dis, bg, wdb, pmat, wf1b, pf, wf2, bf2, out_o):
    d = dis[...][:, 0:1]
    g = _relu(d * (_unpack(a3[...]) + _unpack(tp3[...])) + bg[...])
    h2 = _relu(_mm(g, wdb[...]) + pmat[...])
    f = _relu(_mm(h2, wf1b[...]) + pf[...])
    out_o[...] = _mm(f, wf2[...]) + bf2[...]


_final = pl.pallas_call(
    _final_body,
    grid=_GRID,
    in_specs=[_pack_spec(), _pack_spec(), _row_spec(_DEGW),
              _full_spec(1, _C), _full_spec(_C, _C), _row_spec(_C),
              _full_spec(_C, _C), _row_spec(_C),
              _full_spec(_C, 2), _full_spec(1, 2)],
    out_specs=_row_spec(2),
    out_shape=jax.ShapeDtypeStruct((_N, 2), jnp.float32),
)


@jax.jit
def kernel(x, edge_index, W_pre, b_pre, W_fc1, b_fc1, W_fc2, b_fc2,
           W_g0, b_g0, W_g1, b_g1, W_g2, b_g2,
           W_d0, b_d0, W_d1, b_d1, W_d2, b_d2,
           W_f1, b_f1, W_f2, b_f2):
    src_m = edge_index[0].reshape(_NS, _CH_T, _K)
    dst_m = edge_index[1].reshape(_NS, _CH_T, _K)

    ones_col = jnp.zeros((_K, _DEGW), jnp.float32).at[:, 0].set(1.0)
    zer_deg = jnp.zeros((_RPT, _DEGW), jnp.float32)
    zer_msg = jnp.zeros((_TR, _H), jnp.float32)

    msg_kernel, deg_kernel = _sc_kernels()
    deg4 = deg_kernel(dst_m, ones_col, zer_deg).reshape(_NC, _N, _DEGW)

    r = lambda b: b.reshape(1, -1)
    h2, p0, p1, p2, pf = _prologue(
        x, W_pre, r(b_pre), W_fc1, r(b_fc1), W_fc2, r(b_fc2),
        W_d0[:_C], r(b_d0), W_d1[:_C], r(b_d1), W_d2[:_C], r(b_d2),
        W_f1[:_C], r(b_f1))

    tp3, dis = _scale(h2, deg4, W_g0)

    bg = [b_g0, b_g1, b_g2]
    wdb = [W_d0[_C:], W_d1[_C:], W_d2[_C:]]
    pmat = [p0, p1, p2]
    wgn = [None, W_g1, W_g2]

    v = lambda t: t.reshape(_NC, _NACC, _H)
    pk = lambda a: a.reshape(_NC, _N // 2, _C)

    for i in range(2):
        a3 = pk(msg_kernel(v(tp3), src_m, dst_m, zer_msg))
        tp3 = _layer(a3, tp3, dis, r(bg[i]), wdb[i], pmat[i], wgn[i + 1])

    a3 = pk(msg_kernel(v(tp3), src_m, dst_m, zer_msg))
    out = _final(a3, tp3, dis, r(bg[2]), wdb[2], pmat[2],
                 W_f1[_C:], pf, W_f2, r(b_f2))
    return out
